# baseline (speedup 1.0000x reference)
import functools

import jax
import jax.numpy as jnp
from jax import lax
from jax.experimental import pallas as pl
from jax.experimental.pallas import tpu as pltpu
from jax.experimental.pallas import tpu_sc as plsc

MAP_W = 500
MAP_CELLS = MAP_W * MAP_W
EGO_H, EGO_W, C = 256, 512, 64
TABLE_ROWS = EGO_H * EGO_W
IN_H, IN_W = 120, 160

NC, NS = 2, 16
NW = NC * NS
IDX_MINOR = 128
CHUNK_IDX_ROWS = 8
CHUNK = IDX_MINOR * CHUNK_IDX_ROWS
OUTER = 8
ROWS_PER_WORKER = CHUNK * OUTER
B_PAD = NW * ROWS_PER_WORKER

RB = 2000
N_RBLK = MAP_CELLS // RB

_HIGH = jax.lax.Precision.HIGHEST


def _resize_stage1(a_h, feat_r):
    def body(a_ref, f_ref, o_ref):
        o_ref[...] = jnp.dot(a_ref[...], f_ref[...], precision=_HIGH,
                             preferred_element_type=jnp.float32)

    return pl.pallas_call(
        body,
        out_shape=jax.ShapeDtypeStruct((EGO_H, C * IN_W), jnp.float32),
    )(a_h, feat_r)


def _resize_stage2(a_w, u3):
    BH = 32

    def body(aw_ref, u_ref, o_ref):
        aw = aw_ref[...]
        for h in range(BH):
            x = u_ref[h]
            y = lax.dot_general(aw, x, (((1,), (1,)), ((), ())),
                                precision=_HIGH,
                                preferred_element_type=jnp.float32)
            o_ref[h] = y

    return pl.pallas_call(
        body,
        grid=(EGO_H // BH,),
        in_specs=[
            pl.BlockSpec((EGO_W, IN_W), lambda g: (0, 0)),
            pl.BlockSpec((BH, C, IN_W), lambda g: (g, 0, 0)),
        ],
        out_specs=pl.BlockSpec((BH, EGO_W, C), lambda g: (g, 0, 0)),
        out_shape=jax.ShapeDtypeStruct((EGO_H, EGO_W, C), jnp.float32),
    )(a_w, u3)


def _max_kernel(proj2d):
    def body(p_ref, o_ref):
        o_ref[0, 0] = jnp.max(p_ref[...])

    return pl.pallas_call(
        body,
        out_specs=pl.BlockSpec(memory_space=pltpu.SMEM),
        out_shape=jax.ShapeDtypeStruct((1, 1), jnp.int32),
    )(proj2d)


def _sc_gather(table, idx3):
    mesh = plsc.VectorSubcoreMesh(core_axis_name="c", subcore_axis_name="s",
                                  num_cores=NC, num_subcores=NS)
    idx_rows_per_worker = ROWS_PER_WORKER // IDX_MINOR

    @functools.partial(
        pl.kernel,
        out_type=jax.ShapeDtypeStruct((B_PAD // IDX_MINOR, IDX_MINOR, C),
                                      jnp.float32),
        mesh=mesh,
        scratch_types=[
            pltpu.VMEM((CHUNK_IDX_ROWS, IDX_MINOR), jnp.int32),
            pltpu.VMEM((CHUNK_IDX_ROWS, IDX_MINOR, C), jnp.float32),
            pltpu.SemaphoreType.DMA,
        ],
        compiler_params=pltpu.CompilerParams(use_tc_tiling_on_sc=False),
    )
    def k(table_hbm, idx_hbm, out_hbm, idx_v, rows_v, sem):
        wid = lax.axis_index("s") * NC + lax.axis_index("c")

        def step(i, carry):
            row0 = wid * idx_rows_per_worker + i * CHUNK_IDX_ROWS
            pltpu.sync_copy(idx_hbm.at[pl.ds(row0, CHUNK_IDX_ROWS), :], idx_v)
            descs = [
                pltpu.async_copy(table_hbm.at[idx_v.at[j]], rows_v.at[j], sem)
                for j in range(CHUNK_IDX_ROWS)
            ]
            for d in descs:
                d.wait()
            pltpu.sync_copy(rows_v, out_hbm.at[pl.ds(row0, CHUNK_IDX_ROWS)])
            return carry

        lax.fori_loop(0, OUTER, step, 0)

    return k(table, idx3)


def _transpose_mask(gath, proj3, thr):
    def body(g_ref, p_ref, t_ref, mem_ref, obs_ref):
        x = g_ref[...]
        p = p_ref[0, 0, :]
        m = p < t_ref[0, 0]
        r = lax.broadcasted_iota(jnp.int32, (C, C), 0)
        c = lax.broadcasted_iota(jnp.int32, (C, C), 1)
        eye = (r == c).astype(jnp.float32)
        xt = lax.dot_general(eye, x, (((1,), (1,)), ((), ())),
                             precision=_HIGH,
                             preferred_element_type=jnp.float32)
        mem_ref[:, 0, 0, :] = jnp.where(m[None, :], xt, 0.0)
        obs_ref[0, 0, :] = m.astype(jnp.int8)

    mem, obs = pl.pallas_call(
        body,
        grid=(N_RBLK,),
        in_specs=[
            pl.BlockSpec((RB, C), lambda g: (g, 0)),
            pl.BlockSpec((1, 1, RB), lambda g: (g, 0, 0)),
            pl.BlockSpec(memory_space=pltpu.SMEM),
        ],
        out_specs=[
            pl.BlockSpec((C, 1, 1, RB), lambda g: (0, g, 0, 0)),
            pl.BlockSpec((1, 1, RB), lambda g: (g, 0, 0)),
        ],
        out_shape=[
            jax.ShapeDtypeStruct((C, N_RBLK, 1, RB), jnp.float32),
            jax.ShapeDtypeStruct((N_RBLK, 1, RB), jnp.int8),
        ],
    )(gath, proj3, thr)
    return mem, obs


def kernel(features, proj_indices, masks_inliers):
    del masks_inliers
    f32 = jnp.float32

    a_h = jax.image.resize(jnp.eye(IN_H, dtype=f32), (EGO_H, IN_H),
                           method="bilinear")
    a_w = jax.image.resize(jnp.eye(IN_W, dtype=f32), (EGO_W, IN_W),
                           method="bilinear")

    feat_r = jnp.transpose(features[0], (1, 0, 2)).reshape(IN_H, C * IN_W)
    u = _resize_stage1(a_h, feat_r)
    u3 = u.reshape(EGO_H, C, IN_W)
    table = _resize_stage2(a_w, u3).reshape(TABLE_ROWS, C)

    proj = proj_indices.reshape(-1)
    thr = _max_kernel(proj.reshape(8, MAP_CELLS // 8))

    idx3 = jnp.concatenate(
        [proj, jnp.zeros((B_PAD - MAP_CELLS,), jnp.int32)]
    ).reshape(B_PAD // IDX_MINOR, IDX_MINOR)
    gath = _sc_gather(table, idx3).reshape(B_PAD, C)

    proj3 = proj.reshape(N_RBLK, 1, RB)
    mem, obs = _transpose_mask(gath, proj3, thr)

    memory = mem.reshape(1, C, MAP_W, MAP_W)
    observed = obs.reshape(1, MAP_W, MAP_W).astype(jnp.bool_)
    return memory, observed


# speedup vs baseline: 1.3304x; 1.3304x over previous
import functools

import jax
import jax.numpy as jnp
from jax import lax
from jax.experimental import pallas as pl
from jax.experimental.pallas import tpu as pltpu
from jax.experimental.pallas import tpu_sc as plsc

MAP_W = 500
MAP_CELLS = MAP_W * MAP_W
EGO_H, EGO_W, C = 256, 512, 64
TABLE_ROWS = EGO_H * EGO_W
IN_H, IN_W = 120, 160

NC, NS = 2, 16
NW = NC * NS
IDX_MINOR = 128
CHUNK_IDX_ROWS = 7
CHUNK = IDX_MINOR * CHUNK_IDX_ROWS
OUTER = 9
ROWS_PER_WORKER = CHUNK * OUTER
B_PAD = NW * ROWS_PER_WORKER

RB = 2000
N_RBLK = MAP_CELLS // RB

_HIGH = jax.lax.Precision.HIGHEST


def _resize_stage1(a_h, feat_r):
    def body(a_ref, f_ref, o_ref):
        o_ref[...] = jnp.dot(a_ref[...], f_ref[...], precision=_HIGH,
                             preferred_element_type=jnp.float32)

    return pl.pallas_call(
        body,
        out_shape=jax.ShapeDtypeStruct((EGO_H, C * IN_W), jnp.float32),
    )(a_h, feat_r)


def _resize_stage2(a_w, u3):
    BH = 32

    def body(aw_ref, u_ref, o_ref):
        aw = aw_ref[...]
        for h in range(BH):
            x = u_ref[h]
            y = lax.dot_general(aw, x, (((1,), (1,)), ((), ())),
                                precision=_HIGH,
                                preferred_element_type=jnp.float32)
            o_ref[h] = y

    return pl.pallas_call(
        body,
        grid=(EGO_H // BH,),
        in_specs=[
            pl.BlockSpec((EGO_W, IN_W), lambda g: (0, 0)),
            pl.BlockSpec((BH, C, IN_W), lambda g: (g, 0, 0)),
        ],
        out_specs=pl.BlockSpec((BH, EGO_W, C), lambda g: (g, 0, 0)),
        out_shape=jax.ShapeDtypeStruct((EGO_H, EGO_W, C), jnp.float32),
    )(a_w, u3)


def _max_kernel(proj2d):
    def body(p_ref, o_ref):
        o_ref[0, 0] = jnp.max(p_ref[...])

    return pl.pallas_call(
        body,
        out_specs=pl.BlockSpec(memory_space=pltpu.SMEM),
        out_shape=jax.ShapeDtypeStruct((1, 1), jnp.int32),
    )(proj2d)


def _sc_gather(table, idx3):
    mesh = plsc.VectorSubcoreMesh(core_axis_name="c", subcore_axis_name="s",
                                  num_cores=NC, num_subcores=NS)
    ipw = ROWS_PER_WORKER // IDX_MINOR

    @functools.partial(
        pl.kernel,
        out_type=jax.ShapeDtypeStruct((B_PAD // IDX_MINOR, IDX_MINOR, C),
                                      jnp.float32),
        mesh=mesh,
        scratch_types=[
            pltpu.VMEM((ipw, IDX_MINOR), jnp.int32),
            pltpu.VMEM((CHUNK_IDX_ROWS, IDX_MINOR, C), jnp.float32),
            pltpu.VMEM((CHUNK_IDX_ROWS, IDX_MINOR, C), jnp.float32),
            pltpu.SemaphoreType.DMA,
            pltpu.SemaphoreType.DMA,
            pltpu.SemaphoreType.DMA,
            pltpu.SemaphoreType.DMA,
        ],
        compiler_params=pltpu.CompilerParams(use_tc_tiling_on_sc=False),
    )
    def k(table_hbm, idx_hbm, out_hbm, idx_v, rows0, rows1, gs0, gs1, ss0,
          ss1):
        wid = lax.axis_index("s") * NC + lax.axis_index("c")
        row0 = wid * ipw
        rows, gsem, ssem = [rows0, rows1], [gs0, gs1], [ss0, ss1]

        pltpu.sync_copy(idx_hbm.at[pl.ds(row0, ipw), :], idx_v)

        gd = [None] * OUTER
        sd = [None] * OUTER
        for i in range(OUTER):
            b = i % 2
            if i >= 2:
                sd[i - 2].wait()
            gd[i] = [
                pltpu.async_copy(
                    table_hbm.at[idx_v.at[i * CHUNK_IDX_ROWS + j]],
                    rows[b].at[j], gsem[b])
                for j in range(CHUNK_IDX_ROWS)
            ]
            if i >= 1:
                pb = (i - 1) % 2
                for d in gd[i - 1]:
                    d.wait()
                sd[i - 1] = pltpu.async_copy(
                    rows[pb],
                    out_hbm.at[pl.ds(row0 + (i - 1) * CHUNK_IDX_ROWS,
                                     CHUNK_IDX_ROWS)],
                    ssem[pb])
        last = OUTER - 1
        for d in gd[last]:
            d.wait()
        sd[last] = pltpu.async_copy(
            rows[last % 2],
            out_hbm.at[pl.ds(row0 + last * CHUNK_IDX_ROWS, CHUNK_IDX_ROWS)],
            ssem[last % 2])
        sd[last - 1].wait()
        sd[last].wait()

    return k(table, idx3)


def _transpose_mask(gath, proj3, thr):
    def body(g_ref, p_ref, t_ref, mem_ref, obs_ref):
        x = g_ref[...]
        p = p_ref[0, 0, :]
        m = p < t_ref[0, 0]
        r = lax.broadcasted_iota(jnp.int32, (C, C), 0)
        c = lax.broadcasted_iota(jnp.int32, (C, C), 1)
        eye = (r == c).astype(jnp.float32)
        xt = lax.dot_general(eye, x, (((1,), (1,)), ((), ())),
                             precision=_HIGH,
                             preferred_element_type=jnp.float32)
        mem_ref[:, 0, 0, :] = jnp.where(m[None, :], xt, 0.0)
        obs_ref[0, 0, :] = m.astype(jnp.int8)

    mem, obs = pl.pallas_call(
        body,
        grid=(N_RBLK,),
        in_specs=[
            pl.BlockSpec((RB, C), lambda g: (g, 0)),
            pl.BlockSpec((1, 1, RB), lambda g: (g, 0, 0)),
            pl.BlockSpec(memory_space=pltpu.SMEM),
        ],
        out_specs=[
            pl.BlockSpec((C, 1, 1, RB), lambda g: (0, g, 0, 0)),
            pl.BlockSpec((1, 1, RB), lambda g: (g, 0, 0)),
        ],
        out_shape=[
            jax.ShapeDtypeStruct((C, N_RBLK, 1, RB), jnp.float32),
            jax.ShapeDtypeStruct((N_RBLK, 1, RB), jnp.int8),
        ],
    )(gath, proj3, thr)
    return mem, obs


def kernel(features, proj_indices, masks_inliers):
    del masks_inliers
    f32 = jnp.float32

    a_h = jax.image.resize(jnp.eye(IN_H, dtype=f32), (EGO_H, IN_H),
                           method="bilinear")
    a_w = jax.image.resize(jnp.eye(IN_W, dtype=f32), (EGO_W, IN_W),
                           method="bilinear")

    feat_r = jnp.transpose(features[0], (1, 0, 2)).reshape(IN_H, C * IN_W)
    u = _resize_stage1(a_h, feat_r)
    u3 = u.reshape(EGO_H, C, IN_W)
    table = _resize_stage2(a_w, u3).reshape(TABLE_ROWS, C)

    proj = proj_indices.reshape(-1)
    thr = _max_kernel(proj.reshape(8, MAP_CELLS // 8))

    pad = jnp.arange(B_PAD - MAP_CELLS, dtype=jnp.int32) % TABLE_ROWS
    idx3 = jnp.concatenate([proj, pad]).reshape(B_PAD // IDX_MINOR, IDX_MINOR)
    gath = _sc_gather(table, idx3).reshape(B_PAD, C)

    proj3 = proj.reshape(N_RBLK, 1, RB)
    mem, obs = _transpose_mask(gath, proj3, thr)

    memory = mem.reshape(1, C, MAP_W, MAP_W)
    observed = obs.reshape(1, MAP_W, MAP_W).astype(jnp.bool_)
    return memory, observed


# speedup vs baseline: 2.3688x; 1.7805x over previous
import functools

import jax
import jax.numpy as jnp
from jax import lax
from jax.experimental import pallas as pl
from jax.experimental.pallas import tpu as pltpu
from jax.experimental.pallas import tpu_sc as plsc

MAP_W = 500
MAP_CELLS = MAP_W * MAP_W
EGO_H, EGO_W, C = 256, 512, 64
TABLE_ROWS = EGO_H * EGO_W
IN_H, IN_W = 120, 160

NC, NS = 2, 16
NW = NC * NS
IDX_MINOR = 128
CHUNK_IDX_ROWS = 7
CHUNK = IDX_MINOR * CHUNK_IDX_ROWS
OUTER = 9
ROWS_PER_WORKER = CHUNK * OUTER
B_PAD = NW * ROWS_PER_WORKER

RB = 2000
N_RBLK = MAP_CELLS // RB

_HIGH = jax.lax.Precision.HIGHEST


def _resize_stage1(a_h, feat_r):
    def body(a_ref, f_ref, o_ref):
        o_ref[...] = jnp.dot(a_ref[...], f_ref[...], precision=_HIGH,
                             preferred_element_type=jnp.float32)

    return pl.pallas_call(
        body,
        out_shape=jax.ShapeDtypeStruct((EGO_H, C * IN_W), jnp.float32),
    )(a_h, feat_r)


def _resize_stage2(a_w_even, a_w_odd, u3):
    BH = 32
    W2 = EGO_W // 2

    def body(awe_ref, awo_ref, u_ref, o_ref):
        awe = awe_ref[...]
        awo = awo_ref[...]
        for h in range(BH):
            x = u_ref[h]
            ya = lax.dot_general(awe, x, (((1,), (1,)), ((), ())),
                                 precision=_HIGH,
                                 preferred_element_type=jnp.float32)
            yb = lax.dot_general(awo, x, (((1,), (1,)), ((), ())),
                                 precision=_HIGH,
                                 preferred_element_type=jnp.float32)
            o_ref[h] = jnp.concatenate([ya, yb], axis=1)

    return pl.pallas_call(
        body,
        grid=(EGO_H // BH,),
        in_specs=[
            pl.BlockSpec((W2, IN_W), lambda g: (0, 0)),
            pl.BlockSpec((W2, IN_W), lambda g: (0, 0)),
            pl.BlockSpec((BH, C, IN_W), lambda g: (g, 0, 0)),
        ],
        out_specs=pl.BlockSpec((BH, W2, 2 * C), lambda g: (g, 0, 0)),
        out_shape=jax.ShapeDtypeStruct((EGO_H, W2, 2 * C), jnp.float32),
    )(a_w_even, a_w_odd, u3)


def _max_kernel(proj2d):
    def body(p_ref, o_ref):
        o_ref[0, 0] = jnp.max(p_ref[...])

    return pl.pallas_call(
        body,
        out_specs=pl.BlockSpec(memory_space=pltpu.SMEM),
        out_shape=jax.ShapeDtypeStruct((1, 1), jnp.int32),
    )(proj2d)


def _sc_gather(table, idx3):
    mesh = plsc.VectorSubcoreMesh(core_axis_name="c", subcore_axis_name="s",
                                  num_cores=NC, num_subcores=NS)
    ipw = ROWS_PER_WORKER // IDX_MINOR

    @functools.partial(
        pl.kernel,
        out_type=jax.ShapeDtypeStruct((B_PAD // IDX_MINOR, IDX_MINOR, C),
                                      jnp.float32),
        mesh=mesh,
        scratch_types=[
            pltpu.VMEM((ipw, IDX_MINOR), jnp.int32),
            pltpu.VMEM((CHUNK_IDX_ROWS, IDX_MINOR, C), jnp.float32),
            pltpu.VMEM((CHUNK_IDX_ROWS, IDX_MINOR, C), jnp.float32),
            pltpu.SemaphoreType.DMA,
            pltpu.SemaphoreType.DMA,
            pltpu.SemaphoreType.DMA,
            pltpu.SemaphoreType.DMA,
        ],
        compiler_params=pltpu.CompilerParams(use_tc_tiling_on_sc=False),
    )
    def k(table_hbm, idx_hbm, out_hbm, idx_v, rows0, rows1, gs0, gs1, ss0,
          ss1):
        wid = lax.axis_index("s") * NC + lax.axis_index("c")
        row0 = wid * ipw
        rows, gsem, ssem = [rows0, rows1], [gs0, gs1], [ss0, ss1]

        pltpu.sync_copy(idx_hbm.at[pl.ds(row0, ipw), :], idx_v)

        gd = [None] * OUTER
        sd = [None] * OUTER
        for i in range(OUTER):
            b = i % 2
            if i >= 2:
                sd[i - 2].wait()
            gd[i] = [
                pltpu.async_copy(
                    table_hbm.at[idx_v.at[i * CHUNK_IDX_ROWS + j]],
                    rows[b].at[j], gsem[b])
                for j in range(CHUNK_IDX_ROWS)
            ]
            if i >= 1:
                pb = (i - 1) % 2
                for d in gd[i - 1]:
                    d.wait()
                sd[i - 1] = pltpu.async_copy(
                    rows[pb],
                    out_hbm.at[pl.ds(row0 + (i - 1) * CHUNK_IDX_ROWS,
                                     CHUNK_IDX_ROWS)],
                    ssem[pb])
        last = OUTER - 1
        for d in gd[last]:
            d.wait()
        sd[last] = pltpu.async_copy(
            rows[last % 2],
            out_hbm.at[pl.ds(row0 + last * CHUNK_IDX_ROWS, CHUNK_IDX_ROWS)],
            ssem[last % 2])
        sd[last - 1].wait()
        sd[last].wait()

    return k(table, idx3)


WPAD = 512
N_BLK = MAP_W // 4
HALF = 2 * WPAD


def _transpose_mask(gath2, pblk, thr):
    def body(g_ref, p_ref, t_ref, mem_ref, obs_ref):
        x2 = g_ref[...]
        thv = t_ref[0, 0]
        r = lax.broadcasted_iota(jnp.int32, (C, 2 * C), 0)
        c = lax.broadcasted_iota(jnp.int32, (C, 2 * C), 1)
        ea = (c == r).astype(jnp.float32)
        eb = (c == r + C).astype(jnp.float32)
        nt = (((1,), (1,)), ((), ()))
        xta = lax.dot_general(ea, x2, nt, precision=_HIGH,
                              preferred_element_type=jnp.float32)
        xtb = lax.dot_general(eb, x2, nt, precision=_HIGH,
                              preferred_element_type=jnp.float32)
        for row, (h, xt) in enumerate([(0, xta), (1, xta), (0, xtb),
                                       (1, xtb)]):
            half = 0 if row < 2 else 1
            base = h * WPAD
            m = p_ref[0, half, base:base + MAP_W] < thv
            mem_ref[row] = jnp.where(m[None, :], xt[:, base:base + MAP_W],
                                     0.0)
            mo = (p_ref[0, half, base:base + WPAD] < thv).astype(jnp.int8)
            obs_ref[0, row, :] = mo

    mem, obs = pl.pallas_call(
        body,
        grid=(N_BLK,),
        in_specs=[
            pl.BlockSpec((HALF, 2 * C), lambda g: (g, 0)),
            pl.BlockSpec((1, 2, HALF), lambda g: (g, 0, 0)),
            pl.BlockSpec(memory_space=pltpu.SMEM),
        ],
        out_specs=[
            pl.BlockSpec((4, C, MAP_W), lambda g: (g, 0, 0)),
            pl.BlockSpec((1, 4, WPAD), lambda g: (g, 0, 0)),
        ],
        out_shape=[
            jax.ShapeDtypeStruct((MAP_W, C, MAP_W), jnp.float32),
            jax.ShapeDtypeStruct((N_BLK, 4, WPAD), jnp.int8),
        ],
    )(gath2, pblk, thr)
    return mem, obs


def kernel(features, proj_indices, masks_inliers):
    del masks_inliers
    f32 = jnp.float32

    a_h = jax.image.resize(jnp.eye(IN_H, dtype=f32), (EGO_H, IN_H),
                           method="bilinear")
    a_w = jax.image.resize(jnp.eye(IN_W, dtype=f32), (EGO_W, IN_W),
                           method="bilinear")

    feat_r = jnp.transpose(features[0], (1, 0, 2)).reshape(IN_H, C * IN_W)
    u = _resize_stage1(a_h, feat_r)
    u3 = u.reshape(EGO_H, C, IN_W)
    table = _resize_stage2(a_w[0::2], a_w[1::2], u3).reshape(TABLE_ROWS, C)

    proj = proj_indices.reshape(-1)
    thr = _max_kernel(proj.reshape(8, MAP_CELLS // 8))

    padw = (jnp.arange(MAP_W * (WPAD - MAP_W), dtype=jnp.int32)
            % TABLE_ROWS).reshape(MAP_W, WPAD - MAP_W)
    projp = jnp.concatenate([proj.reshape(MAP_W, MAP_W), padw], axis=1)
    pblk = projp.reshape(N_BLK, 2, HALF)
    idx_perm = jnp.transpose(pblk, (0, 2, 1)).reshape(-1)
    tail = jnp.arange(B_PAD - idx_perm.size, dtype=jnp.int32) % TABLE_ROWS
    idx3 = jnp.concatenate([idx_perm, tail]).reshape(B_PAD // IDX_MINOR,
                                                     IDX_MINOR)
    gath2 = _sc_gather(table, idx3).reshape(B_PAD // 2, 2 * C)

    mem, obs = _transpose_mask(gath2, pblk, thr)

    memory = jnp.transpose(mem, (1, 0, 2)).reshape(1, C, MAP_W, MAP_W)
    observed = (obs.reshape(MAP_W, WPAD)[:, :MAP_W]
                .reshape(1, MAP_W, MAP_W).astype(jnp.bool_))
    return memory, observed


# speedup vs baseline: 3.1202x; 1.3172x over previous
import functools

import jax
import jax.numpy as jnp
from jax import lax
from jax.experimental import pallas as pl
from jax.experimental.pallas import tpu as pltpu
from jax.experimental.pallas import tpu_sc as plsc

MAP_W = 500
MAP_CELLS = MAP_W * MAP_W
EGO_H, EGO_W, C = 256, 512, 64
TABLE_ROWS = EGO_H * EGO_W
IN_H, IN_W = 120, 160

NC, NS = 2, 16
NW = NC * NS
IDX_MINOR = 128
CHUNK_IDX_ROWS = 7
CHUNK = IDX_MINOR * CHUNK_IDX_ROWS
OUTER = 9
ROWS_PER_WORKER = CHUNK * OUTER
B_PAD = NW * ROWS_PER_WORKER

RB = 2000
N_RBLK = MAP_CELLS // RB

_HIGH = jax.lax.Precision.HIGHEST


def _resize_dot1(feat2, a_w):
    def body(f_ref, a_ref, o_ref):
        o_ref[...] = lax.dot_general(f_ref[...], a_ref[...],
                                     (((1,), (1,)), ((), ())),
                                     precision=_HIGH,
                                     preferred_element_type=jnp.float32)

    NB = 4
    RB1 = IN_H * C // NB
    return pl.pallas_call(
        body,
        grid=(NB,),
        in_specs=[
            pl.BlockSpec((RB1, IN_W), lambda g: (g, 0)),
            pl.BlockSpec((EGO_W, IN_W), lambda g: (0, 0)),
        ],
        out_specs=pl.BlockSpec((RB1, EGO_W), lambda g: (g, 0)),
        out_shape=jax.ShapeDtypeStruct((IN_H * C, EGO_W), jnp.float32),
    )(feat2, a_w)


def _resize_dot2(a_h, v3):
    NB = 4
    CB = EGO_W * C // NB

    def body(a_ref, v_ref, o_ref):
        o_ref[...] = lax.dot_general(a_ref[...], v_ref[...],
                                     (((1,), (0,)), ((), ())),
                                     precision=_HIGH,
                                     preferred_element_type=jnp.float32)

    return pl.pallas_call(
        body,
        grid=(NB,),
        in_specs=[
            pl.BlockSpec((EGO_H, IN_H), lambda g: (0, 0)),
            pl.BlockSpec((IN_H, CB), lambda g: (0, g)),
        ],
        out_specs=pl.BlockSpec((EGO_H, CB), lambda g: (0, g)),
        out_shape=jax.ShapeDtypeStruct((EGO_H, EGO_W * C), jnp.float32),
    )(a_h, v3)


def _max_kernel(proj2d):
    def body(p_ref, o_ref):
        o_ref[0, 0] = jnp.max(p_ref[...])

    return pl.pallas_call(
        body,
        out_specs=pl.BlockSpec(memory_space=pltpu.SMEM),
        out_shape=jax.ShapeDtypeStruct((1, 1), jnp.int32),
    )(proj2d)


def _sc_gather(table, idx3):
    mesh = plsc.VectorSubcoreMesh(core_axis_name="c", subcore_axis_name="s",
                                  num_cores=NC, num_subcores=NS)
    ipw = ROWS_PER_WORKER // IDX_MINOR

    @functools.partial(
        pl.kernel,
        out_type=jax.ShapeDtypeStruct((B_PAD // IDX_MINOR, IDX_MINOR, C),
                                      jnp.float32),
        mesh=mesh,
        scratch_types=[
            pltpu.VMEM((ipw, IDX_MINOR), jnp.int32),
            pltpu.VMEM((CHUNK_IDX_ROWS, IDX_MINOR, C), jnp.float32),
            pltpu.VMEM((CHUNK_IDX_ROWS, IDX_MINOR, C), jnp.float32),
            pltpu.SemaphoreType.DMA,
            pltpu.SemaphoreType.DMA,
            pltpu.SemaphoreType.DMA,
            pltpu.SemaphoreType.DMA,
        ],
        compiler_params=pltpu.CompilerParams(use_tc_tiling_on_sc=False),
    )
    def k(table_hbm, idx_hbm, out_hbm, idx_v, rows0, rows1, gs0, gs1, ss0,
          ss1):
        wid = lax.axis_index("s") * NC + lax.axis_index("c")
        row0 = wid * ipw
        rows, gsem, ssem = [rows0, rows1], [gs0, gs1], [ss0, ss1]

        pltpu.sync_copy(idx_hbm.at[pl.ds(row0, ipw), :], idx_v)

        gd = [None] * OUTER
        sd = [None] * OUTER
        for i in range(OUTER):
            b = i % 2
            if i >= 2:
                sd[i - 2].wait()
            gd[i] = [
                pltpu.async_copy(
                    table_hbm.at[idx_v.at[i * CHUNK_IDX_ROWS + j]],
                    rows[b].at[j], gsem[b])
                for j in range(CHUNK_IDX_ROWS)
            ]
            if i >= 1:
                pb = (i - 1) % 2
                for d in gd[i - 1]:
                    d.wait()
                sd[i - 1] = pltpu.async_copy(
                    rows[pb],
                    out_hbm.at[pl.ds(row0 + (i - 1) * CHUNK_IDX_ROWS,
                                     CHUNK_IDX_ROWS)],
                    ssem[pb])
        last = OUTER - 1
        for d in gd[last]:
            d.wait()
        sd[last] = pltpu.async_copy(
            rows[last % 2],
            out_hbm.at[pl.ds(row0 + last * CHUNK_IDX_ROWS, CHUNK_IDX_ROWS)],
            ssem[last % 2])
        sd[last - 1].wait()
        sd[last].wait()

    return k(table, idx3)


WPAD = 512
N_BLK = MAP_W // 4
HALF = 2 * WPAD


def _transpose_mask(gath2, pblk, thr):
    def body(g_ref, p_ref, t_ref, mem_ref, obs_ref):
        x2 = g_ref[...]
        thv = t_ref[0, 0]
        xt = jnp.transpose(x2)
        for row in range(4):
            half, h = row // 2, row % 2
            base = h * WPAD
            c0 = half * C
            m = p_ref[0, half, base:base + MAP_W] < thv
            mem_ref[row] = jnp.where(m[None, :],
                                     xt[c0:c0 + C, base:base + MAP_W], 0.0)
            mo = (p_ref[0, half, base:base + WPAD] < thv).astype(jnp.int8)
            obs_ref[0, row, :] = mo

    mem, obs = pl.pallas_call(
        body,
        grid=(N_BLK,),
        in_specs=[
            pl.BlockSpec((HALF, 2 * C), lambda g: (g, 0)),
            pl.BlockSpec((1, 2, HALF), lambda g: (g, 0, 0)),
            pl.BlockSpec(memory_space=pltpu.SMEM),
        ],
        out_specs=[
            pl.BlockSpec((4, C, MAP_W), lambda g: (g, 0, 0)),
            pl.BlockSpec((1, 4, WPAD), lambda g: (g, 0, 0)),
        ],
        out_shape=[
            jax.ShapeDtypeStruct((MAP_W, C, MAP_W), jnp.float32),
            jax.ShapeDtypeStruct((N_BLK, 4, WPAD), jnp.int8),
        ],
    )(gath2, pblk, thr)
    return mem, obs


def kernel(features, proj_indices, masks_inliers):
    del masks_inliers
    f32 = jnp.float32

    a_h = jax.image.resize(jnp.eye(IN_H, dtype=f32), (EGO_H, IN_H),
                           method="bilinear")
    a_w = jax.image.resize(jnp.eye(IN_W, dtype=f32), (EGO_W, IN_W),
                           method="bilinear")

    feat2 = jnp.transpose(features[0], (1, 0, 2)).reshape(IN_H * C, IN_W)
    v2 = _resize_dot1(feat2, a_w)
    v3 = (v2.reshape(IN_H, C, EGO_W).transpose(0, 2, 1)
          .reshape(IN_H, EGO_W * C))
    table = _resize_dot2(a_h, v3).reshape(TABLE_ROWS, C)

    proj = proj_indices.reshape(-1)
    thr = _max_kernel(proj.reshape(8, MAP_CELLS // 8))

    padw = (jnp.arange(MAP_W * (WPAD - MAP_W), dtype=jnp.int32)
            % TABLE_ROWS).reshape(MAP_W, WPAD - MAP_W)
    projp = jnp.concatenate([proj.reshape(MAP_W, MAP_W), padw], axis=1)
    pblk = projp.reshape(N_BLK, 2, HALF)
    idx_perm = jnp.transpose(pblk, (0, 2, 1)).reshape(-1)
    tail = jnp.arange(B_PAD - idx_perm.size, dtype=jnp.int32) % TABLE_ROWS
    idx3 = jnp.concatenate([idx_perm, tail]).reshape(B_PAD // IDX_MINOR,
                                                     IDX_MINOR)
    gath2 = _sc_gather(table, idx3).reshape(B_PAD // 2, 2 * C)

    mem, obs = _transpose_mask(gath2, pblk, thr)

    memory = jnp.transpose(mem, (1, 0, 2)).reshape(1, C, MAP_W, MAP_W)
    observed = (obs.reshape(MAP_W, WPAD)[:, :MAP_W]
                .reshape(1, MAP_W, MAP_W).astype(jnp.bool_))
    return memory, observed


# speedup vs baseline: 3.9018x; 1.2505x over previous
import functools

import jax
import jax.numpy as jnp
from jax import lax
from jax.experimental import pallas as pl
from jax.experimental.pallas import tpu as pltpu
from jax.experimental.pallas import tpu_sc as plsc

MAP_W = 500
MAP_CELLS = MAP_W * MAP_W
EGO_H, EGO_W, C = 256, 512, 64
TABLE_ROWS = EGO_H * EGO_W
IN_H, IN_W = 120, 160

NC, NS = 2, 16
NW = NC * NS
IDX_MINOR = 128
CHUNK_IDX_ROWS = 4
CHUNK = IDX_MINOR * CHUNK_IDX_ROWS
OUTER = 16
ROWS_PER_WORKER = CHUNK * OUTER
B_PAD = NW * ROWS_PER_WORKER

_HIGH = jax.lax.Precision.HIGHEST


def _resize_dot1(feat2, a_w):
    def body(f_ref, a_ref, o_ref):
        o_ref[...] = lax.dot_general(f_ref[...], a_ref[...],
                                     (((1,), (1,)), ((), ())),
                                     precision=_HIGH,
                                     preferred_element_type=jnp.float32)

    NB = 4
    RB1 = IN_H * C // NB
    return pl.pallas_call(
        body,
        grid=(NB,),
        in_specs=[
            pl.BlockSpec((RB1, IN_W), lambda g: (g, 0)),
            pl.BlockSpec((EGO_W, IN_W), lambda g: (0, 0)),
        ],
        out_specs=pl.BlockSpec((RB1, EGO_W), lambda g: (g, 0)),
        out_shape=jax.ShapeDtypeStruct((IN_H * C, EGO_W), jnp.float32),
    )(feat2, a_w)


def _resize_dot2(a_h, v3):
    NB = 4
    CB = EGO_W * C // NB

    def body(a_ref, v_ref, o_ref):
        o_ref[...] = lax.dot_general(a_ref[...], v_ref[...],
                                     (((1,), (0,)), ((), ())),
                                     precision=_HIGH,
                                     preferred_element_type=jnp.float32)

    return pl.pallas_call(
        body,
        grid=(NB,),
        in_specs=[
            pl.BlockSpec((EGO_H, IN_H), lambda g: (0, 0)),
            pl.BlockSpec((IN_H, CB), lambda g: (0, g)),
        ],
        out_specs=pl.BlockSpec((EGO_H, CB), lambda g: (0, g)),
        out_shape=jax.ShapeDtypeStruct((EGO_H, EGO_W * C), jnp.float32),
    )(a_h, v3)


def _max_kernel(proj2d):
    def body(p_ref, o_ref):
        o_ref[0, 0] = jnp.max(p_ref[...])

    return pl.pallas_call(
        body,
        out_specs=pl.BlockSpec(memory_space=pltpu.SMEM),
        out_shape=jax.ShapeDtypeStruct((1, 1), jnp.int32),
    )(proj2d)


def _sc_gather(table, idx1d):
    mesh = plsc.VectorSubcoreMesh(core_axis_name="c", subcore_axis_name="s",
                                  num_cores=NC, num_subcores=NS)
    ipw = ROWS_PER_WORKER // IDX_MINOR

    @functools.partial(
        pl.kernel,
        out_type=jax.ShapeDtypeStruct((B_PAD // IDX_MINOR, IDX_MINOR, C),
                                      jnp.float32),
        mesh=mesh,
        scratch_types=[
            pltpu.VMEM((ROWS_PER_WORKER,), jnp.int32),
            pltpu.VMEM((ipw, IDX_MINOR), jnp.int32),
            pltpu.VMEM((CHUNK_IDX_ROWS, IDX_MINOR, C), jnp.float32),
            pltpu.VMEM((CHUNK_IDX_ROWS, IDX_MINOR, C), jnp.float32),
            pltpu.SemaphoreType.DMA,
            pltpu.SemaphoreType.DMA,
            pltpu.SemaphoreType.DMA,
            pltpu.SemaphoreType.DMA,
        ],
        compiler_params=pltpu.CompilerParams(use_tc_tiling_on_sc=False,
                                             needs_layout_passes=False),
    )
    def k(table_hbm, idx_hbm, out_hbm, nat_v, idx_v, rows0, rows1, gs0, gs1,
          ss0, ss1):
        wid = lax.axis_index("s") * NC + lax.axis_index("c")
        row0 = wid * ipw
        rows, gsem, ssem = [rows0, rows1], [gs0, gs1], [ss0, ss1]

        pltpu.sync_copy(idx_hbm.at[pl.ds(wid * ROWS_PER_WORKER,
                                         ROWS_PER_WORKER)], nat_v)

        lane = lax.iota(jnp.int32, 16)
        t16 = lane >> 1
        par = (lane & 1) << 10

        def perm_row(r, carry):
            for cc in range(IDX_MINOR // 16):
                q = r * 64 + cc * 8 + t16
                cell = ((q >> 10) << 11) + (q & 1023) + par
                vals = plsc.load_gather(nat_v, [cell])
                idx_v[r, pl.ds(cc * 16, 16)] = vals
            return carry

        lax.fori_loop(0, ipw, perm_row, 0)

        gd = [None] * OUTER
        sd = [None] * OUTER
        for i in range(OUTER):
            b = i % 2
            if i >= 2:
                sd[i - 2].wait()
            gd[i] = [
                pltpu.async_copy(
                    table_hbm.at[idx_v.at[i * CHUNK_IDX_ROWS + j]],
                    rows[b].at[j], gsem[b])
                for j in range(CHUNK_IDX_ROWS)
            ]
            if i >= 1:
                pb = (i - 1) % 2
                for d in gd[i - 1]:
                    d.wait()
                sd[i - 1] = pltpu.async_copy(
                    rows[pb],
                    out_hbm.at[pl.ds(row0 + (i - 1) * CHUNK_IDX_ROWS,
                                     CHUNK_IDX_ROWS)],
                    ssem[pb])
        last = OUTER - 1
        for d in gd[last]:
            d.wait()
        sd[last] = pltpu.async_copy(
            rows[last % 2],
            out_hbm.at[pl.ds(row0 + last * CHUNK_IDX_ROWS, CHUNK_IDX_ROWS)],
            ssem[last % 2])
        sd[last - 1].wait()
        sd[last].wait()

    return k(table, idx1d)


WPAD = 512
N_BLK = MAP_W // 4
HALF = 2 * WPAD


def _transpose_mask(gath2, pblk, thr):
    def body(g_ref, p_ref, t_ref, mem_ref, obs_ref):
        x2 = g_ref[...]
        thv = t_ref[0, 0]
        xt = jnp.transpose(x2)
        for row in range(4):
            half, h = row // 2, row % 2
            base = h * WPAD
            c0 = half * C
            m = p_ref[0, half, base:base + MAP_W] < thv
            mem_ref[row] = jnp.where(m[None, :],
                                     xt[c0:c0 + C, base:base + MAP_W], 0.0)
            mo = (p_ref[0, half, base:base + WPAD] < thv).astype(jnp.int8)
            obs_ref[0, row, :] = mo

    mem, obs = pl.pallas_call(
        body,
        grid=(N_BLK,),
        in_specs=[
            pl.BlockSpec((HALF, 2 * C), lambda g: (g, 0)),
            pl.BlockSpec((1, 2, HALF), lambda g: (g, 0, 0)),
            pl.BlockSpec(memory_space=pltpu.SMEM),
        ],
        out_specs=[
            pl.BlockSpec((4, C, MAP_W), lambda g: (g, 0, 0)),
            pl.BlockSpec((1, 4, WPAD), lambda g: (g, 0, 0)),
        ],
        out_shape=[
            jax.ShapeDtypeStruct((MAP_W, C, MAP_W), jnp.float32),
            jax.ShapeDtypeStruct((N_BLK, 4, WPAD), jnp.int8),
        ],
    )(gath2, pblk, thr)
    return mem, obs


def kernel(features, proj_indices, masks_inliers):
    del masks_inliers
    f32 = jnp.float32

    a_h = jax.image.resize(jnp.eye(IN_H, dtype=f32), (EGO_H, IN_H),
                           method="bilinear")
    a_w = jax.image.resize(jnp.eye(IN_W, dtype=f32), (EGO_W, IN_W),
                           method="bilinear")

    feat2 = features[0].reshape(C * IN_H, IN_W)
    v2 = _resize_dot1(feat2, a_w)
    v3 = (v2.reshape(C, IN_H, EGO_W).transpose(1, 2, 0)
          .reshape(IN_H, EGO_W * C))
    table = _resize_dot2(a_h, v3).reshape(TABLE_ROWS, C)

    proj = proj_indices.reshape(-1)
    thr = _max_kernel(proj.reshape(8, MAP_CELLS // 8))

    padw = (jnp.arange(MAP_W * (WPAD - MAP_W), dtype=jnp.int32)
            % TABLE_ROWS).reshape(MAP_W, WPAD - MAP_W)
    projp = jnp.concatenate([proj.reshape(MAP_W, MAP_W), padw], axis=1)
    pblk = projp.reshape(N_BLK, 2, HALF)
    tail = jnp.arange(B_PAD - projp.size, dtype=jnp.int32) % TABLE_ROWS
    idx1d = jnp.concatenate([projp.reshape(-1), tail])
    gath2 = _sc_gather(table, idx1d).reshape(B_PAD // 2, 2 * C)

    mem, obs = _transpose_mask(gath2, pblk, thr)

    memory = jnp.transpose(mem, (1, 0, 2)).reshape(1, C, MAP_W, MAP_W)
    observed = (obs.reshape(MAP_W, WPAD)[:, :MAP_W]
                .reshape(1, MAP_W, MAP_W).astype(jnp.bool_))
    return memory, observed


# speedup vs baseline: 4.7071x; 1.2064x over previous
import functools

import jax
import jax.numpy as jnp
from jax import lax
from jax.experimental import pallas as pl
from jax.experimental.pallas import tpu as pltpu
from jax.experimental.pallas import tpu_sc as plsc

MAP_W = 500
MAP_CELLS = MAP_W * MAP_W
EGO_H, EGO_W, C = 256, 512, 64
TABLE_ROWS = EGO_H * EGO_W
IN_H, IN_W = 120, 160

NC, NS = 2, 16
NW = NC * NS
IDX_MINOR = 128
CHUNK_IDX_ROWS = 4
CHUNK = IDX_MINOR * CHUNK_IDX_ROWS
OUTER = 16
ROWS_PER_WORKER = CHUNK * OUTER
B_PAD = NW * ROWS_PER_WORKER

_HIGH = jax.lax.Precision.HIGHEST


def _resize_dot1(feat2, a_w):
    def body(f_ref, a_ref, o_ref):
        o_ref[...] = lax.dot_general(f_ref[...], a_ref[...],
                                     (((1,), (1,)), ((), ())),
                                     precision=_HIGH,
                                     preferred_element_type=jnp.float32)

    NB = 4
    RB1 = IN_H * C // NB
    return pl.pallas_call(
        body,
        grid=(NB,),
        in_specs=[
            pl.BlockSpec((RB1, IN_W), lambda g: (g, 0)),
            pl.BlockSpec((EGO_W, IN_W), lambda g: (0, 0)),
        ],
        out_specs=pl.BlockSpec((RB1, EGO_W), lambda g: (g, 0)),
        out_shape=jax.ShapeDtypeStruct((IN_H * C, EGO_W), jnp.float32),
    )(feat2, a_w)


def _resize_dot2(a_h, v3):
    NB = 4
    CB = EGO_W * C // NB

    def body(a_ref, v_ref, o_ref):
        o_ref[...] = lax.dot_general(a_ref[...], v_ref[...],
                                     (((1,), (0,)), ((), ())),
                                     precision=_HIGH,
                                     preferred_element_type=jnp.float32)

    return pl.pallas_call(
        body,
        grid=(NB,),
        in_specs=[
            pl.BlockSpec((EGO_H, IN_H), lambda g: (0, 0)),
            pl.BlockSpec((IN_H, CB), lambda g: (0, g)),
        ],
        out_specs=pl.BlockSpec((EGO_H, CB), lambda g: (0, g)),
        out_shape=jax.ShapeDtypeStruct((EGO_H, EGO_W * C), jnp.float32),
    )(a_h, v3)


def _max_kernel(proj2d):
    def body(p_ref, o_ref):
        o_ref[0, 0] = jnp.max(p_ref[...])

    return pl.pallas_call(
        body,
        out_specs=pl.BlockSpec(memory_space=pltpu.SMEM),
        out_shape=jax.ShapeDtypeStruct((1, 1), jnp.int32),
    )(proj2d)


def _sc_gather(table, idx1d):
    mesh = plsc.VectorSubcoreMesh(core_axis_name="c", subcore_axis_name="s",
                                  num_cores=NC, num_subcores=NS)
    ipw = ROWS_PER_WORKER // IDX_MINOR

    @functools.partial(
        pl.kernel,
        out_type=jax.ShapeDtypeStruct((B_PAD // IDX_MINOR, IDX_MINOR, C),
                                      jnp.float32),
        mesh=mesh,
        scratch_types=[
            pltpu.VMEM((ROWS_PER_WORKER,), jnp.int32),
            pltpu.VMEM((ipw, IDX_MINOR), jnp.int32),
            pltpu.VMEM((CHUNK_IDX_ROWS, IDX_MINOR, C), jnp.float32),
            pltpu.VMEM((CHUNK_IDX_ROWS, IDX_MINOR, C), jnp.float32),
            pltpu.SemaphoreType.DMA,
            pltpu.SemaphoreType.DMA,
            pltpu.SemaphoreType.DMA,
            pltpu.SemaphoreType.DMA,
        ],
        compiler_params=pltpu.CompilerParams(use_tc_tiling_on_sc=False,
                                             needs_layout_passes=False),
    )
    def k(table_hbm, idx_hbm, out_hbm, nat_v, idx_v, rows0, rows1, gs0, gs1,
          ss0, ss1):
        wid = lax.axis_index("s") * NC + lax.axis_index("c")
        row0 = wid * ipw
        rows, gsem, ssem = [rows0, rows1], [gs0, gs1], [ss0, ss1]

        pltpu.sync_copy(idx_hbm.at[pl.ds(wid * ROWS_PER_WORKER,
                                         ROWS_PER_WORKER)], nat_v)

        lane = lax.iota(jnp.int32, 16)
        t16 = lane >> 1
        par = (lane & 1) << 10

        def perm_row(r, carry):
            for cc in range(IDX_MINOR // 16):
                q = r * 64 + cc * 8 + t16
                cell = ((q >> 10) << 11) + (q & 1023) + par
                vals = plsc.load_gather(nat_v, [cell])
                idx_v[r, pl.ds(cc * 16, 16)] = vals
            return carry

        lax.fori_loop(0, ipw, perm_row, 0)

        gd = [None] * OUTER
        sd = [None] * OUTER
        for i in range(OUTER):
            b = i % 2
            if i >= 2:
                sd[i - 2].wait()
            gd[i] = [
                pltpu.async_copy(
                    table_hbm.at[idx_v.at[i * CHUNK_IDX_ROWS + j]],
                    rows[b].at[j], gsem[b])
                for j in range(CHUNK_IDX_ROWS)
            ]
            if i >= 1:
                pb = (i - 1) % 2
                for d in gd[i - 1]:
                    d.wait()
                sd[i - 1] = pltpu.async_copy(
                    rows[pb],
                    out_hbm.at[pl.ds(row0 + (i - 1) * CHUNK_IDX_ROWS,
                                     CHUNK_IDX_ROWS)],
                    ssem[pb])
        last = OUTER - 1
        for d in gd[last]:
            d.wait()
        sd[last] = pltpu.async_copy(
            rows[last % 2],
            out_hbm.at[pl.ds(row0 + last * CHUNK_IDX_ROWS, CHUNK_IDX_ROWS)],
            ssem[last % 2])
        sd[last - 1].wait()
        sd[last].wait()

    return k(table, idx1d)


WPAD = 512
N_BLK = MAP_W // 4
HALF = 2 * WPAD


def _transpose_mask(gath2, pblk, thr):
    SB = 5

    def body(g_ref, p_ref, t_ref, mem_ref, obs_ref):
        x2 = g_ref[...]
        thv = t_ref[0, 0]
        xt = jnp.transpose(x2)
        for sb in range(SB):
            for row in range(4):
                half, h = row // 2, row % 2
                base = h * WPAD
                c0 = half * C
                off = sb * HALF + base
                m = p_ref[sb, half, base:base + MAP_W] < thv
                mem_ref[4 * sb + row] = jnp.where(
                    m[None, :], xt[c0:c0 + C, off:off + MAP_W], 0.0)
                mo = (p_ref[sb, half, base:base + WPAD] < thv
                      ).astype(jnp.int8)
                obs_ref[sb, row, :] = mo

    mem, obs = pl.pallas_call(
        body,
        grid=(N_BLK // SB,),
        in_specs=[
            pl.BlockSpec((SB * HALF, 2 * C), lambda g: (g, 0)),
            pl.BlockSpec((SB, 2, HALF), lambda g: (g, 0, 0)),
            pl.BlockSpec(memory_space=pltpu.SMEM),
        ],
        out_specs=[
            pl.BlockSpec((4 * SB, C, MAP_W), lambda g: (g, 0, 0)),
            pl.BlockSpec((SB, 4, WPAD), lambda g: (g, 0, 0)),
        ],
        out_shape=[
            jax.ShapeDtypeStruct((MAP_W, C, MAP_W), jnp.float32),
            jax.ShapeDtypeStruct((N_BLK, 4, WPAD), jnp.int8),
        ],
    )(gath2, pblk, thr)
    return mem, obs


def kernel(features, proj_indices, masks_inliers):
    del masks_inliers
    f32 = jnp.float32

    a_h = jax.image.resize(jnp.eye(IN_H, dtype=f32), (EGO_H, IN_H),
                           method="bilinear")
    a_w = jax.image.resize(jnp.eye(IN_W, dtype=f32), (EGO_W, IN_W),
                           method="bilinear")

    feat2 = features[0].reshape(C * IN_H, IN_W)
    v2 = _resize_dot1(feat2, a_w)
    v3 = (v2.reshape(C, IN_H, EGO_W).transpose(1, 2, 0)
          .reshape(IN_H, EGO_W * C))
    table = _resize_dot2(a_h, v3).reshape(TABLE_ROWS, C)

    proj = proj_indices.reshape(-1)
    thr = _max_kernel(proj.reshape(8, MAP_CELLS // 8))

    padw = (jnp.arange(MAP_W * (WPAD - MAP_W), dtype=jnp.int32)
            % TABLE_ROWS).reshape(MAP_W, WPAD - MAP_W)
    projp = jnp.concatenate([proj.reshape(MAP_W, MAP_W), padw], axis=1)
    pblk = projp.reshape(N_BLK, 2, HALF)
    tail = jnp.arange(B_PAD - projp.size, dtype=jnp.int32) % TABLE_ROWS
    idx1d = jnp.concatenate([projp.reshape(-1), tail])
    gath2 = _sc_gather(table, idx1d).reshape(B_PAD // 2, 2 * C)

    mem, obs = _transpose_mask(gath2, pblk, thr)

    memory = jnp.transpose(mem, (1, 0, 2)).reshape(1, C, MAP_W, MAP_W)
    observed = (obs.reshape(MAP_W, WPAD)[:, :MAP_W]
                .reshape(1, MAP_W, MAP_W).astype(jnp.bool_))
    return memory, observed
